# Initial kernel scaffold; baseline (speedup 1.0000x reference)
import jax
import jax.numpy as jnp
from jax.experimental import pallas as pl


def kernel(feat, id, W, b, table):
    raise NotImplementedError("write your pallas kernel here")


# speedup vs baseline: 1.0137x; 1.0137x over previous
import functools

import jax
import jax.numpy as jnp
from jax import lax
from jax.experimental import pallas as pl
from jax.experimental.pallas import tpu as pltpu
from jax.experimental.pallas import tpu_sc as plsc

N_NODES = 50000
D_FEAT = 256
HIDDEN = 128

NUM_CORES = 2
NUM_SUBCORES = 16
NW = NUM_CORES * NUM_SUBCORES

B_PAD = 50176
B_PER_W = B_PAD // NW
CHUNK = 112
N_CHUNKS = B_PER_W // CHUNK

@functools.cache
def _make_sc_gather():
    mesh = plsc.VectorSubcoreMesh(core_axis_name="c", subcore_axis_name="s")
    return functools.partial(
        pl.kernel,
        mesh=mesh,
        out_type=jax.ShapeDtypeStruct((B_PAD, HIDDEN), jnp.float32),
        scratch_types=[
            pltpu.VMEM((CHUNK,), jnp.int32),
            pltpu.VMEM((CHUNK,), jnp.int32),
            pltpu.VMEM((CHUNK, HIDDEN), jnp.float32),
            pltpu.VMEM((CHUNK, HIDDEN), jnp.float32),
            pltpu.SemaphoreType.DMA,
            pltpu.SemaphoreType.DMA,
        ],
    )(_sc_gather_body)


def _sc_gather_body(
    table_hbm, idx_hbm, out_hbm, idx0, idx1, rows0, rows1, sem0, sem1
):
    wid = lax.axis_index("s") * NUM_CORES + lax.axis_index("c")
    base = wid * B_PER_W
    idxs = (idx0, idx1)
    bufs = (rows0, rows1)
    sems = (sem0, sem1)

    def start(c):
        s = c % 2
        pltpu.sync_copy(idx_hbm.at[pl.ds(base + c * CHUNK, CHUNK)], idxs[s])
        return pltpu.async_copy(table_hbm.at[idxs[s]], bufs[s], sems[s])

    cps = [start(0), start(1)]
    for c in range(N_CHUNKS):
        s = c % 2
        cps[s].wait()
        pltpu.sync_copy(bufs[s], out_hbm.at[pl.ds(base + c * CHUNK, CHUNK)])
        if c + 2 < N_CHUNKS:
            cps[s] = start(c + 2)


BR = 2000


def _mm_body(feat_ref, w_ref, b_ref, g_ref, out_ref):
    out_ref[...] = (
        jnp.dot(feat_ref[...], w_ref[...], preferred_element_type=jnp.float32)
        + b_ref[...]
        + g_ref[...]
    )


def kernel(feat, id, W, b, table):
    ids = id.astype(jnp.int32)
    ids_pad = jnp.pad(ids, (0, B_PAD - N_NODES))
    g = _make_sc_gather()(table, ids_pad)
    out = pl.pallas_call(
        _mm_body,
        grid=(N_NODES // BR,),
        in_specs=[
            pl.BlockSpec((BR, D_FEAT), lambda i: (i, 0)),
            pl.BlockSpec((D_FEAT, HIDDEN), lambda i: (0, 0)),
            pl.BlockSpec((1, HIDDEN), lambda i: (0, 0)),
            pl.BlockSpec((BR, HIDDEN), lambda i: (i, 0)),
        ],
        out_specs=pl.BlockSpec((BR, HIDDEN), lambda i: (i, 0)),
        out_shape=jax.ShapeDtypeStruct((N_NODES, HIDDEN), jnp.float32),
    )(feat, W, b.reshape(1, HIDDEN), g)
    return out
